# baseline (speedup 1.0000x reference)
import functools

import jax
import jax.numpy as jnp
from jax import lax
from jax.experimental import pallas as pl
from jax.experimental.pallas import tpu as pltpu
from jax.experimental.pallas import tpu_sc as plsc

_INFO = plsc.get_sparse_core_info()
_NC = _INFO.num_cores
_NS = _INFO.num_subcores
_NW = _NC * _NS
_RING = 64
_NBUF = 6


def _make_gather(B, V, D):
    full_cols = (V - 1) // 128
    tail_lo = full_cols * 128
    tail_w = V - tail_lo
    cpw = -(-full_cols // _NW)
    tail_base = cpw * 128
    mesh = plsc.VectorSubcoreMesh(core_axis_name="c", subcore_axis_name="s")

    @functools.partial(
        pl.kernel,
        mesh=mesh,
        out_type=jax.ShapeDtypeStruct((B, D), jnp.float32),
        scratch_types=[
            pltpu.VMEM((B,), jnp.int32),
            pltpu.VMEM((B + 16,), jnp.int32),
            pltpu.VMEM((B + 16,), jnp.int32),
            pltpu.VMEM((D, 128 * _NBUF), jnp.float32),
            pltpu.VMEM((D, tail_w), jnp.float32),
            pltpu.VMEM((_RING, D), jnp.float32),
            pltpu.SMEM((cpw + 2,), jnp.int32),
            pltpu.SMEM((cpw + 2,), jnp.int32),
            pltpu.SMEM((cpw + 2,), jnp.int32),
            pltpu.SMEM((cpw + 2,), jnp.int32),
            pltpu.SemaphoreType.DMA((_NBUF,)),
            pltpu.SemaphoreType.DMA,
        ],
        compiler_params=pltpu.CompilerParams(
            use_tc_tiling_on_sc=True, needs_layout_passes=False
        ),
    )
    def gather_kernel(tableT, idx_hbm, out_hbm, idx_all, mpk, mpk_o, win,
                      tailb, ring, hist, offs, cur, wlist, wsem, rsem):
        wid = lax.axis_index("s") * _NC + lax.axis_index("c")
        lo_col = wid * cpw
        ncol = jnp.minimum(cpw, full_cols - lo_col)
        lo = lo_col * 128
        hi = lo + ncol * 128
        iota = lax.iota(jnp.int32, 16)

        def issue(t):
            half = pl.multiple_of(lax.rem(t, _NBUF) * 128, 128)
            src_c = pl.multiple_of((lo_col + wlist[t]) * 128, 128)
            pltpu.async_copy(
                tableT.at[:, pl.ds(src_c, 128)],
                win.at[:, pl.ds(half, 128)],
                wsem.at[lax.rem(t, _NBUF)],
            )

        for i in range(_NBUF):
            wlist[i] = jnp.minimum(jnp.int32(i), ncol - 1)
            @pl.when(i < ncol)
            def _(i=i):
                issue(jnp.int32(i))

        pltpu.sync_copy(idx_hbm, idx_all)
        pltpu.sync_copy(tableT.at[:, pl.ds(tail_lo, tail_w)], tailb)

        iotash = iota << 15

        def scan_body(g, off):
            vec = idx_all[pl.ds(g * 16, 16)]
            m = (vec >= lo) & (vec < hi)
            cnt = plsc.all_reduce_population_count(m)[0]

            @pl.when(cnt > 0)
            def _():
                packed = (vec - lo) | (iotash + (g << 19))
                plsc.store_compressed(mpk.at[pl.ds(off, 16)], packed, mask=m)

            return off + cnt

        off0 = lax.fori_loop(0, B // 16, scan_body, jnp.int32(0),
                             unroll=8)

        def tscan_body(g, off):
            vec = idx_all[pl.ds(g * 16, 16)]
            m = vec >= tail_lo
            cnt = plsc.all_reduce_population_count(m)[0]
            packed = (vec - tail_lo + tail_base) | ((iota + g * 16) << 15)
            plsc.store_compressed(mpk.at[pl.ds(off, 16)], packed, mask=m)
            return off + cnt

        g0 = wid * (B // (16 * _NW))
        n_w = lax.fori_loop(g0, g0 + B // (16 * _NW), tscan_body, off0,
                            unroll=False)

        def zero_body(i, _):
            hist[i] = 0
            return 0

        lax.fori_loop(0, ncol + 1, zero_body, 0, unroll=False)

        def bin_of(pk):
            return jnp.minimum((pk & 32767) >> 7, ncol)

        def hist_body(g, _):
            vec = mpk[pl.ds(g * 16, 16)]
            for j in range(16):
                @pl.when(g * 16 + j < n_w)
                def _(vec=vec, j=j):
                    b = bin_of(vec[j])
                    hist[b] = hist[b] + 1
            return 0

        lax.fori_loop(0, (n_w + 15) // 16, hist_body, 0, unroll=False)

        def pfx_body(i, run):
            c = hist[i]
            offs[i] = run
            cur[i] = run
            return run + c

        lax.fori_loop(0, ncol + 1, pfx_body, jnp.int32(0), unroll=False)

        lane0 = iota == 0

        def ord_body(g, _):
            vec = mpk[pl.ds(g * 16, 16)]
            for j in range(16):
                @pl.when(g * 16 + j < n_w)
                def _(vec=vec, j=j):
                    pk = vec[j]
                    b = bin_of(pk)
                    s = cur[b]
                    cur[b] = s + 1
                    plsc.store_scatter(
                        mpk_o, [jnp.full((16,), s, jnp.int32)],
                        jnp.full((16,), pk, jnp.int32), mask=lane0)
            return 0

        lax.fori_loop(0, (n_w + 15) // 16, ord_body, 0, unroll=False)

        def build_body(i, m):
            wlist[m] = i
            return m + jnp.where(hist[i] > 0, 1, 0)

        nw2 = lax.fori_loop(jnp.minimum(jnp.int32(_NBUF), ncol), ncol,
                            build_body,
                            jnp.minimum(jnp.int32(_NBUF), ncol),
                            unroll=False)

        def serve(buf, colofs, base, n_t, fired0):
            def match_body(k, fired):
                vec = mpk_o[pl.ds(base + k, 16)]
                pk = vec[0]
                b = pk >> 15
                lr = (pk & 32767) - colofs
                slot = lax.rem(fired, _RING)

                @pl.when(fired >= _RING)
                def _():
                    pltpu.make_async_copy(
                        ring.at[pl.ds(0, 1)], out_hbm.at[pl.ds(0, 1)], rsem
                    ).wait()

                lrv = jnp.full((16,), lr, jnp.int32)
                sv = jnp.full((16,), slot, jnp.int32)
                for q in range(D // 16):
                    vals = plsc.load_gather(buf, [iota + q * 16, lrv])
                    plsc.store_scatter(ring, [sv, iota + q * 16], vals)
                pltpu.async_copy(
                    ring.at[pl.ds(slot, 1)], out_hbm.at[pl.ds(b, 1)], rsem
                )
                return fired + 1

            return lax.fori_loop(0, n_t, match_body, fired0, unroll=False)

        def win_loop(t, fired):
            pltpu.make_async_copy(
                tableT.at[:, pl.ds(0, 128)],
                win.at[:, pl.ds(pl.multiple_of(0, 128), 128)],
                wsem.at[lax.rem(t, _NBUF)],
            ).wait()

            c = wlist[t]
            colofs = c * 128 - lax.rem(t, _NBUF) * 128
            fired = serve(win, colofs, offs[c], hist[c], fired)

            @pl.when(t + _NBUF < nw2)
            def _():
                issue(t + _NBUF)

            return fired

        fired = lax.fori_loop(0, nw2, win_loop, jnp.int32(0), unroll=False)
        fired = serve(tailb, tail_base, offs[ncol], hist[ncol], fired)

        def drain_body(k, _):
            pltpu.make_async_copy(
                ring.at[pl.ds(0, 1)], out_hbm.at[pl.ds(0, 1)], rsem
            ).wait()
            return 0

        lax.fori_loop(0, jnp.minimum(fired, _RING), drain_body, 0,
                      unroll=False)

    return gather_kernel


def kernel(cond_ids, batch_size, table):
    B = cond_ids.shape[0]
    V, D = table.shape
    return _make_gather(B, V, D)(table.T, cond_ids.astype(jnp.int32))


# speedup vs baseline: 1.0596x; 1.0596x over previous
import functools

import jax
import jax.numpy as jnp
from jax import lax
from jax.experimental import pallas as pl
from jax.experimental.pallas import tpu as pltpu
from jax.experimental.pallas import tpu_sc as plsc

_INFO = plsc.get_sparse_core_info()
_NC = _INFO.num_cores
_NS = _INFO.num_subcores
_NW = _NC * _NS
_RING = 64
_NBUF = 6


def _make_gather(B, V, D):
    full_cols = (V - 1) // 128
    tail_lo = full_cols * 128
    tail_w = V - tail_lo
    cpw = -(-full_cols // _NW)
    tail_base = cpw * 128
    mesh = plsc.VectorSubcoreMesh(core_axis_name="c", subcore_axis_name="s")

    @functools.partial(
        pl.kernel,
        mesh=mesh,
        out_type=jax.ShapeDtypeStruct((B, D), jnp.float32),
        scratch_types=[
            pltpu.VMEM((B,), jnp.int32),
            pltpu.VMEM((B + 16,), jnp.int32),
            pltpu.VMEM((B + 16,), jnp.int32),
            pltpu.VMEM((D, 128 * _NBUF), jnp.float32),
            pltpu.VMEM((D, tail_w), jnp.float32),
            pltpu.VMEM((_RING, D), jnp.float32),
            pltpu.SMEM((cpw + 2,), jnp.int32),
            pltpu.SMEM((cpw + 2,), jnp.int32),
            pltpu.SMEM((cpw + 2,), jnp.int32),
            pltpu.SMEM((cpw + 2,), jnp.int32),
            pltpu.SemaphoreType.DMA((_NBUF,)),
            pltpu.SemaphoreType.DMA,
        ],
        compiler_params=pltpu.CompilerParams(
            use_tc_tiling_on_sc=True, needs_layout_passes=False
        ),
    )
    def gather_kernel(tableT, idx_hbm, out_hbm, idx_all, mpk, mpk_o, win,
                      tailb, ring, hist, offs, cur, wlist, wsem, rsem):
        wid = lax.axis_index("s") * _NC + lax.axis_index("c")
        lo_col = wid * cpw
        ncol = jnp.minimum(cpw, full_cols - lo_col)
        lo = lo_col * 128
        hi = lo + ncol * 128
        iota = lax.iota(jnp.int32, 16)

        def issue(t):
            half = pl.multiple_of(lax.rem(t, _NBUF) * 128, 128)
            src_c = pl.multiple_of((lo_col + wlist[t]) * 128, 128)
            pltpu.async_copy(
                tableT.at[:, pl.ds(src_c, 128)],
                win.at[:, pl.ds(half, 128)],
                wsem.at[lax.rem(t, _NBUF)],
            )

        pltpu.sync_copy(idx_hbm, idx_all)
        pltpu.sync_copy(tableT.at[:, pl.ds(tail_lo, tail_w)], tailb)

        iotash = iota << 15

        def scan_body(g, off):
            vec = idx_all[pl.ds(g * 16, 16)]
            m = (vec >= lo) & (vec < hi)
            cnt = plsc.all_reduce_population_count(m)[0]
            packed = (vec - lo) | (iotash + (g << 19))
            plsc.store_compressed(mpk.at[pl.ds(off, 16)], packed, mask=m)
            return off + cnt

        off0 = lax.fori_loop(0, B // 16, scan_body, jnp.int32(0),
                             unroll=8)

        def tscan_body(g, off):
            vec = idx_all[pl.ds(g * 16, 16)]
            m = vec >= tail_lo
            cnt = plsc.all_reduce_population_count(m)[0]
            packed = (vec - tail_lo + tail_base) | ((iota + g * 16) << 15)
            plsc.store_compressed(mpk.at[pl.ds(off, 16)], packed, mask=m)
            return off + cnt

        g0 = wid * (B // (16 * _NW))
        n_w = lax.fori_loop(g0, g0 + B // (16 * _NW), tscan_body, off0,
                            unroll=False)

        def zero_body(i, _):
            hist[i] = 0
            return 0

        lax.fori_loop(0, ncol + 1, zero_body, 0, unroll=False)

        def bin_of(pk):
            return jnp.minimum((pk & 32767) >> 7, ncol)

        def hist_body(g, _):
            vec = mpk[pl.ds(g * 16, 16)]
            for j in range(16):
                @pl.when(g * 16 + j < n_w)
                def _(vec=vec, j=j):
                    b = bin_of(vec[j])
                    hist[b] = hist[b] + 1
            return 0

        lax.fori_loop(0, (n_w + 15) // 16, hist_body, 0, unroll=False)

        def pfx_body(i, run):
            c = hist[i]
            offs[i] = run
            cur[i] = run
            return run + c

        lax.fori_loop(0, ncol + 1, pfx_body, jnp.int32(0), unroll=False)

        lane0 = iota == 0

        def ord_body(g, _):
            vec = mpk[pl.ds(g * 16, 16)]
            for j in range(16):
                @pl.when(g * 16 + j < n_w)
                def _(vec=vec, j=j):
                    pk = vec[j]
                    b = bin_of(pk)
                    s = cur[b]
                    cur[b] = s + 1
                    plsc.store_scatter(
                        mpk_o, [jnp.full((16,), s, jnp.int32)],
                        jnp.full((16,), pk, jnp.int32), mask=lane0)
            return 0

        lax.fori_loop(0, (n_w + 15) // 16, ord_body, 0, unroll=False)

        def build_body(i, m):
            wlist[m] = i
            return m + jnp.where(hist[i] > 0, 1, 0)

        nw2 = lax.fori_loop(0, ncol, build_body, jnp.int32(0), unroll=False)

        for i in range(_NBUF):
            @pl.when(i < nw2)
            def _(i=i):
                issue(jnp.int32(i))

        def serve(buf, colofs, base, n_t, fired0):
            def match_body(k, fired):
                vec = mpk_o[pl.ds(base + k, 16)]
                pk = vec[0]
                b = pk >> 15
                lr = (pk & 32767) - colofs
                slot = lax.rem(fired, _RING)

                @pl.when(fired >= _RING)
                def _():
                    pltpu.make_async_copy(
                        ring.at[pl.ds(0, 1)], out_hbm.at[pl.ds(0, 1)], rsem
                    ).wait()

                lrv = jnp.full((16,), lr, jnp.int32)
                sv = jnp.full((16,), slot, jnp.int32)
                for q in range(D // 16):
                    vals = plsc.load_gather(buf, [iota + q * 16, lrv])
                    plsc.store_scatter(ring, [sv, iota + q * 16], vals)
                pltpu.async_copy(
                    ring.at[pl.ds(slot, 1)], out_hbm.at[pl.ds(b, 1)], rsem
                )
                return fired + 1

            return lax.fori_loop(0, n_t, match_body, fired0, unroll=False)

        def win_loop(t, fired):
            pltpu.make_async_copy(
                tableT.at[:, pl.ds(0, 128)],
                win.at[:, pl.ds(pl.multiple_of(0, 128), 128)],
                wsem.at[lax.rem(t, _NBUF)],
            ).wait()

            c = wlist[t]
            colofs = c * 128 - lax.rem(t, _NBUF) * 128
            fired = serve(win, colofs, offs[c], hist[c], fired)

            @pl.when(t + _NBUF < nw2)
            def _():
                issue(t + _NBUF)

            return fired

        fired = lax.fori_loop(0, nw2, win_loop, jnp.int32(0), unroll=False)
        fired = serve(tailb, tail_base, offs[ncol], hist[ncol], fired)

        def drain_body(k, _):
            pltpu.make_async_copy(
                ring.at[pl.ds(0, 1)], out_hbm.at[pl.ds(0, 1)], rsem
            ).wait()
            return 0

        lax.fori_loop(0, jnp.minimum(fired, _RING), drain_body, 0,
                      unroll=False)

    return gather_kernel


def kernel(cond_ids, batch_size, table):
    B = cond_ids.shape[0]
    V, D = table.shape
    return _make_gather(B, V, D)(table.T, cond_ids.astype(jnp.int32))
